# baseline (speedup 1.0000x reference)
import functools

import jax
import jax.numpy as jnp
from jax import lax
from jax.experimental import pallas as pl
from jax.experimental.pallas import tpu as pltpu
from jax.experimental.pallas import tpu_sc as plsc

_BATCH = 4096
_HIST = 200
_DIM = 64
_B = _BATCH * _HIST
_NW = 32
_B_PER_W = _B // _NW
_CHUNK = 1024
_N_CHUNKS = _B_PER_W // _CHUNK


def _emb_body(idx_hbm, table_hbm, out_hbm, idx_v, rows_v, sem):
    wid = lax.axis_index("s") * 2 + lax.axis_index("c")
    base = wid * _B_PER_W

    def step(c, _):
        start = base + c * _CHUNK
        pltpu.sync_copy(idx_hbm.at[pl.ds(start, _CHUNK)], idx_v)
        pltpu.async_copy(table_hbm.at[idx_v], rows_v, sem).wait()
        pltpu.sync_copy(rows_v, out_hbm.at[pl.ds(start, _CHUNK)])
        return ()

    lax.fori_loop(0, _N_CHUNKS, step, (), unroll=False)


_mesh = plsc.VectorSubcoreMesh(core_axis_name="c", subcore_axis_name="s")

_emb = functools.partial(
    pl.kernel,
    out_type=jax.ShapeDtypeStruct((_B, _DIM), jnp.float32),
    mesh=_mesh,
    scratch_types=[
        pltpu.VMEM((_CHUNK,), jnp.int32),
        pltpu.VMEM((_CHUNK, _DIM), jnp.float32),
        pltpu.SemaphoreType.DMA,
    ],
    compiler_params=pltpu.CompilerParams(use_tc_tiling_on_sc=False),
)(_emb_body)


@jax.jit
def kernel(inputs, table):
    idx = inputs.reshape(_B)
    out = _emb(idx, table)
    return out.reshape(_BATCH, _HIST, _DIM)


# speedup vs baseline: 1.0114x; 1.0114x over previous
import functools

import jax
import jax.numpy as jnp
from jax import lax
from jax.experimental import pallas as pl
from jax.experimental.pallas import tpu as pltpu
from jax.experimental.pallas import tpu_sc as plsc

_BATCH = 4096
_HIST = 200
_DIM = 64
_B = _BATCH * _HIST
_NW = 32
_B_PER_W = _B // _NW
_CHUNK = 800
_N_CHUNKS = _B_PER_W // _CHUNK
_N_ROUNDS = _N_CHUNKS // 2


def _emb_body(idx_hbm, table_hbm, out_hbm,
              idx_a, idx_b, rows_a, rows_b, gs_a, gs_b, ws_a, ws_b):
    wid = lax.axis_index("s") * 2 + lax.axis_index("c")
    base = wid * _B_PER_W

    def round_(i, _):
        a = base + (2 * i) * _CHUNK
        b = a + _CHUNK

        @pl.when(i > 0)
        def _():
            pltpu.make_async_copy(rows_a, out_hbm.at[pl.ds(0, _CHUNK)], ws_a).wait()
            pltpu.make_async_copy(rows_b, out_hbm.at[pl.ds(0, _CHUNK)], ws_b).wait()

        pltpu.sync_copy(idx_hbm.at[pl.ds(a, _CHUNK)], idx_a)
        ga = pltpu.async_copy(table_hbm.at[idx_a], rows_a, gs_a)
        pltpu.sync_copy(idx_hbm.at[pl.ds(b, _CHUNK)], idx_b)
        gb = pltpu.async_copy(table_hbm.at[idx_b], rows_b, gs_b)
        ga.wait()
        pltpu.async_copy(rows_a, out_hbm.at[pl.ds(a, _CHUNK)], ws_a)
        gb.wait()
        pltpu.async_copy(rows_b, out_hbm.at[pl.ds(b, _CHUNK)], ws_b)
        return ()

    lax.fori_loop(0, _N_ROUNDS, round_, (), unroll=False)
    pltpu.make_async_copy(rows_a, out_hbm.at[pl.ds(0, _CHUNK)], ws_a).wait()
    pltpu.make_async_copy(rows_b, out_hbm.at[pl.ds(0, _CHUNK)], ws_b).wait()


_mesh = plsc.VectorSubcoreMesh(core_axis_name="c", subcore_axis_name="s")

_emb = functools.partial(
    pl.kernel,
    out_type=jax.ShapeDtypeStruct((_B, _DIM), jnp.float32),
    mesh=_mesh,
    scratch_types=[
        pltpu.VMEM((_CHUNK,), jnp.int32),
        pltpu.VMEM((_CHUNK,), jnp.int32),
        pltpu.VMEM((_CHUNK, _DIM), jnp.float32),
        pltpu.VMEM((_CHUNK, _DIM), jnp.float32),
        pltpu.SemaphoreType.DMA,
        pltpu.SemaphoreType.DMA,
        pltpu.SemaphoreType.DMA,
        pltpu.SemaphoreType.DMA,
    ],
    compiler_params=pltpu.CompilerParams(use_tc_tiling_on_sc=False),
)(_emb_body)


@jax.jit
def kernel(inputs, table):
    idx = inputs.reshape(_B)
    out = _emb(idx, table)
    return out.reshape(_BATCH, _HIST, _DIM)


# speedup vs baseline: 1.0164x; 1.0050x over previous
import functools

import jax
import jax.numpy as jnp
from jax import lax
from jax.experimental import pallas as pl
from jax.experimental.pallas import tpu as pltpu
from jax.experimental.pallas import tpu_sc as plsc

_BATCH = 4096
_HIST = 200
_DIM = 64
_B = _BATCH * _HIST
_NW = 32
_B_PER_W = _B // _NW
_CHUNK = 400
_NC = _B_PER_W // _CHUNK
_NBUF = 4
_ROUNDS = _NC // _NBUF
_LAG = 2


def _emb_body(idx_hbm, table_hbm, out_hbm, idx_v,
              rows0, rows1, rows2, rows3, g0, g1, g2, g3, w0, w1, w2, w3):
    rows = (rows0, rows1, rows2, rows3)
    gsem = (g0, g1, g2, g3)
    wsem = (w0, w1, w2, w3)
    wid = lax.axis_index("s") * 2 + lax.axis_index("c")
    base = wid * _B_PER_W
    brow = wid * (_B_PER_W // _HIST)

    pltpu.sync_copy(idx_hbm.at[pl.ds(base, _B_PER_W)], idx_v)

    def gather_start(s, c):
        for h in range(2):
            pltpu.async_copy(
                table_hbm.at[idx_v.at[pl.ds(c * _CHUNK + h * _HIST, _HIST)]],
                rows[s].at[h], gsem[s])

    def gather_wait(s):
        pltpu.make_async_copy(
            out_hbm.at[pl.ds(0, 2)], rows[s], gsem[s]).wait()

    def wb_start(s, c):
        pltpu.async_copy(
            rows[s], out_hbm.at[pl.ds(brow + 2 * c, 2)], wsem[s])

    def wb_wait(s):
        pltpu.make_async_copy(
            rows[s], out_hbm.at[pl.ds(0, 2)], wsem[s]).wait()

    def round_(i, _):
        for s in range(_NBUF):
            c = _NBUF * i + s
            d = c - _LAG
            t = (s + _NBUF - _LAG) % _NBUF

            @pl.when(c >= _NBUF)
            def _():
                wb_wait(s)

            gather_start(s, c)

            @pl.when(d >= 0)
            def _():
                gather_wait(t)
                wb_start(t, d)
        return ()

    lax.fori_loop(0, _ROUNDS, round_, (), unroll=False)

    for c in (_NC - _LAG, _NC - 1):
        gather_wait(c % _NBUF)
        wb_start(c % _NBUF, c)
    for s in range(_NBUF):
        wb_wait(s)


_mesh = plsc.VectorSubcoreMesh(core_axis_name="c", subcore_axis_name="s")

_emb = functools.partial(
    pl.kernel,
    out_type=jax.ShapeDtypeStruct((_BATCH, _HIST, _DIM), jnp.float32),
    mesh=_mesh,
    scratch_types=[
        pltpu.VMEM((_B_PER_W,), jnp.int32),
        pltpu.VMEM((2, _HIST, _DIM), jnp.float32),
        pltpu.VMEM((2, _HIST, _DIM), jnp.float32),
        pltpu.VMEM((2, _HIST, _DIM), jnp.float32),
        pltpu.VMEM((2, _HIST, _DIM), jnp.float32),
        pltpu.SemaphoreType.DMA,
        pltpu.SemaphoreType.DMA,
        pltpu.SemaphoreType.DMA,
        pltpu.SemaphoreType.DMA,
        pltpu.SemaphoreType.DMA,
        pltpu.SemaphoreType.DMA,
        pltpu.SemaphoreType.DMA,
        pltpu.SemaphoreType.DMA,
    ],
    compiler_params=pltpu.CompilerParams(use_tc_tiling_on_sc=False),
)(_emb_body)


@jax.jit
def kernel(inputs, table):
    idx = inputs.reshape(_B)
    return _emb(idx, table)


# speedup vs baseline: 1.2396x; 1.2196x over previous
import functools

import jax
import jax.numpy as jnp
from jax import lax
from jax.experimental import pallas as pl
from jax.experimental.pallas import tpu as pltpu
from jax.experimental.pallas import tpu_sc as plsc

_BATCH = 4096
_HIST = 200
_DIM = 64
_PAD = 128
_B = _BATCH * _HIST
_NW = 32
_B_PER_W = _B // _NW
_CHUNK = _HIST
_NC = _B_PER_W // _CHUNK
_NBUF = 2
_ROUNDS = _NC // _NBUF
_LAG = 1


def _emb_body(idx_hbm, table_hbm, out_hbm, idx_v, rows0, rows1,
              g0, g1, w0, w1):
    rows = (rows0, rows1)
    gsem = (g0, g1)
    wsem = (w0, w1)
    wid = lax.axis_index("s") * 2 + lax.axis_index("c")
    base = wid * _B_PER_W
    brow = wid * (_B_PER_W // _HIST)

    pltpu.sync_copy(idx_hbm.at[pl.ds(base, _B_PER_W)], idx_v)

    def gather_start(s, c):
        pltpu.async_copy(
            table_hbm.at[idx_v.at[pl.ds(c * _CHUNK, _CHUNK)]],
            rows[s], gsem[s])

    def gather_wait(s):
        pltpu.make_async_copy(
            table_hbm.at[idx_v.at[pl.ds(0, _CHUNK)]], rows[s], gsem[s]).wait()

    def wb_start(s, c):
        pltpu.async_copy(rows[s], out_hbm.at[brow + c], wsem[s])

    def wb_wait(s):
        pltpu.make_async_copy(rows[s], out_hbm.at[0], wsem[s]).wait()

    def round_(i, _):
        for s in range(_NBUF):
            c = _NBUF * i + s
            d = c - _LAG
            t = (s + _NBUF - _LAG) % _NBUF

            @pl.when(c >= _NBUF)
            def _():
                wb_wait(s)

            gather_start(s, c)

            @pl.when(d >= 0)
            def _():
                gather_wait(t)
                wb_start(t, d)
        return ()

    lax.fori_loop(0, _ROUNDS, round_, (), unroll=False)

    for c in range(_NC - _LAG, _NC):
        gather_wait(c % _NBUF)
        wb_start(c % _NBUF, c)
    for s in range(_NBUF):
        wb_wait(s)


_mesh = plsc.VectorSubcoreMesh(core_axis_name="c", subcore_axis_name="s")

_emb = functools.partial(
    pl.kernel,
    out_type=jax.ShapeDtypeStruct((_BATCH, _HIST, _PAD), jnp.float32),
    mesh=_mesh,
    scratch_types=[
        pltpu.VMEM((_B_PER_W,), jnp.int32),
        pltpu.VMEM((_CHUNK, _PAD), jnp.float32),
        pltpu.VMEM((_CHUNK, _PAD), jnp.float32),
        pltpu.SemaphoreType.DMA,
        pltpu.SemaphoreType.DMA,
        pltpu.SemaphoreType.DMA,
        pltpu.SemaphoreType.DMA,
    ],
    compiler_params=pltpu.CompilerParams(use_tc_tiling_on_sc=True),
)(_emb_body)


@jax.jit
def kernel(inputs, table):
    idx = inputs.reshape(_B)
    tablep = jnp.pad(table, ((0, 0), (0, _PAD - _DIM)))
    return _emb(idx, tablep)[:, :, :_DIM]
